# baseline (speedup 1.0000x reference)
import jax
import jax.numpy as jnp
from jax import lax
from jax.experimental import pallas as pl
from jax.experimental.pallas import tpu as pltpu
from jax.experimental.pallas import tpu_sc as plsc

_L = 16


def _tracker_update_body(trk_hbm, head_hbm, cmp_hbm, out_hbm,
                         head_v, b0, b1, c0, c1,
                         sem_head, sl0, sl1, sc0, sc1, ss0, ss1):
    num_sel = head_hbm.shape[0]
    width = cmp_hbm.shape[1]
    wid = lax.axis_index("s") * 2 + lax.axis_index("c")

    bufs = (b0, b1)
    sem_ld = (sl0, sl1)
    sem_st = (ss0, ss1)
    cmp_bufs = (c0, c1)
    sem_cmp = (sc0, sc1)

    loads = [pltpu.async_copy(trk_hbm.at[2 * wid + rr], bufs[rr], sem_ld[rr])
             for rr in range(2)]
    pltpu.async_copy(head_hbm, head_v, sem_head).wait()

    sel = [jnp.bool_(False), jnp.bool_(False)]
    j = [jnp.int32(0), jnp.int32(0)]
    for c in range(num_sel // _L):
        hv = head_v[pl.ds(c * _L, _L)]
        for i in range(_L):
            h = hv[i]
            for rr in range(2):
                hit = h == 2 * wid + rr
                sel[rr] = sel[rr] | hit
                j[rr] = jnp.where(hit, jnp.int32(c * _L + i), j[rr])

    for rr in range(2):
        @pl.when(sel[rr])
        def _(rr=rr):
            pltpu.async_copy(cmp_hbm.at[j[rr]], cmp_bufs[rr], sem_cmp[rr])

    neg1 = jnp.full((_L,), -1, jnp.int32)
    for rr in range(2):
        loads[rr].wait()
        r = 2 * wid + rr

        @pl.when(sel[rr])
        def _(rr=rr, r=r):
            pltpu.async_copy(bufs[rr].at[pl.ds(width, width)],
                             out_hbm.at[r, pl.ds(width, width)], sem_st[rr])
            pltpu.make_async_copy(cmp_hbm.at[j[rr]], cmp_bufs[rr],
                                  sem_cmp[rr]).wait()

            def mask_body(k, carry):
                bs = k * _L
                cv = cmp_bufs[rr][pl.ds(bs, _L)]
                tv = bufs[rr][pl.ds(bs, _L)]
                bufs[rr][pl.ds(bs, _L)] = jnp.where(cv != -1, neg1, tv)
                return carry

            lax.fori_loop(0, width // _L, mask_body, 0, unroll=16)
            pltpu.async_copy(bufs[rr].at[pl.ds(0, width)],
                             out_hbm.at[r, pl.ds(0, width)], sem_st[rr])

        @pl.when(jnp.logical_not(sel[rr]))
        def _(rr=rr, r=r):
            pltpu.async_copy(bufs[rr], out_hbm.at[r], sem_st[rr])

    for rr in range(2):
        pltpu.make_async_copy(bufs[rr], out_hbm.at[2 * wid + rr],
                              sem_st[rr]).wait()


def kernel(tracker, head_idx, seq_idx, compute_idx):
    num_heads, row_len = tracker.shape
    num_sel, width = compute_idx.shape
    del seq_idx

    kern = pl.kernel(
        _tracker_update_body,
        out_type=jax.ShapeDtypeStruct((num_heads, row_len), jnp.int32),
        mesh=plsc.VectorSubcoreMesh(core_axis_name="c", subcore_axis_name="s"),
        scratch_types=[
            pltpu.VMEM((num_sel,), jnp.int32),
            pltpu.VMEM((row_len,), jnp.int32),
            pltpu.VMEM((row_len,), jnp.int32),
            pltpu.VMEM((width,), jnp.int32),
            pltpu.VMEM((width,), jnp.int32),
        ] + [pltpu.SemaphoreType.DMA] * 7,
    )
    return kern(tracker, head_idx, compute_idx)


# speedup vs baseline: 1.2792x; 1.2792x over previous
import jax
import jax.numpy as jnp
from jax import lax
from jax.experimental import pallas as pl
from jax.experimental.pallas import tpu as pltpu
from jax.experimental.pallas import tpu_sc as plsc

_L = 16


def _tracker_update_body(trk_hbm, head_hbm, cmp_hbm, out_hbm,
                         head_v, b0, b1, c0, c1,
                         sem_head, sl0, sl1, sc0, sc1, ss0, ss1):
    num_sel = head_hbm.shape[0]
    width = cmp_hbm.shape[1]
    wid = lax.axis_index("s") * 2 + lax.axis_index("c")

    bufs = (b0, b1)
    sem_ld = (sl0, sl1)
    sem_st = (ss0, ss1)
    cmp_bufs = (c0, c1)
    sem_cmp = (sc0, sc1)

    loads = [pltpu.async_copy(trk_hbm.at[2 * wid + rr], bufs[rr], sem_ld[rr])
             for rr in range(2)]
    pltpu.async_copy(head_hbm, head_v, sem_head).wait()

    sel = [jnp.bool_(False), jnp.bool_(False)]
    j = [jnp.int32(0), jnp.int32(0)]
    for c in range(num_sel // _L):
        hv = head_v[pl.ds(c * _L, _L)]
        for i in range(_L):
            h = hv[i]
            for rr in range(2):
                hit = h == 2 * wid + rr
                sel[rr] = sel[rr] | hit
                j[rr] = jnp.where(hit, jnp.int32(c * _L + i), j[rr])

    for rr in range(2):
        @pl.when(sel[rr])
        def _(rr=rr):
            pltpu.async_copy(cmp_hbm.at[j[rr]], cmp_bufs[rr], sem_cmp[rr])

    neg1 = jnp.full((_L,), -1, jnp.int32)
    for rr in range(2):
        loads[rr].wait()
        r = 2 * wid + rr

        @pl.when(sel[rr])
        def _(rr=rr, r=r):
            pltpu.async_copy(bufs[rr].at[pl.ds(width, width)],
                             out_hbm.at[r, pl.ds(width, width)], sem_st[rr])
            pltpu.make_async_copy(cmp_hbm.at[j[rr]], cmp_bufs[rr],
                                  sem_cmp[rr]).wait()

            @plsc.parallel_loop(0, width, step=_L, unroll=8)
            def mask_body(bs):
                cv = cmp_bufs[rr][pl.ds(bs, _L)]
                tv = bufs[rr][pl.ds(bs, _L)]
                bufs[rr][pl.ds(bs, _L)] = jnp.where(cv != -1, neg1, tv)
            pltpu.async_copy(bufs[rr].at[pl.ds(0, width)],
                             out_hbm.at[r, pl.ds(0, width)], sem_st[rr])

        @pl.when(jnp.logical_not(sel[rr]))
        def _(rr=rr, r=r):
            pltpu.async_copy(bufs[rr], out_hbm.at[r], sem_st[rr])

    for rr in range(2):
        pltpu.make_async_copy(bufs[rr], out_hbm.at[2 * wid + rr],
                              sem_st[rr]).wait()


def kernel(tracker, head_idx, seq_idx, compute_idx):
    num_heads, row_len = tracker.shape
    num_sel, width = compute_idx.shape
    del seq_idx

    kern = pl.kernel(
        _tracker_update_body,
        out_type=jax.ShapeDtypeStruct((num_heads, row_len), jnp.int32),
        mesh=plsc.VectorSubcoreMesh(core_axis_name="c", subcore_axis_name="s"),
        scratch_types=[
            pltpu.VMEM((num_sel,), jnp.int32),
            pltpu.VMEM((row_len,), jnp.int32),
            pltpu.VMEM((row_len,), jnp.int32),
            pltpu.VMEM((width,), jnp.int32),
            pltpu.VMEM((width,), jnp.int32),
        ] + [pltpu.SemaphoreType.DMA] * 7,
    )
    return kern(tracker, head_idx, compute_idx)
